# baseline (speedup 1.0000x reference)
import functools

import jax
import jax.numpy as jnp
from jax import lax
from jax.experimental import pallas as pl
from jax.experimental.pallas import tpu as pltpu
from jax.experimental.pallas import tpu_sc as plsc

N = 10000
D = 128
E = 320000

NC = 2
NS = 16
NTILES = NC * NS
CHUNK = 80
NCH = 125
EPT = NCH * CHUNK
NPAD = 10240
RPT = NPAD // NS


def _sc_segsum_body(h_hbm, pidx_hbm, out_hbm,
                    pidx_v, ring, rows0, rows1, acc, sem0, sem1):
    c = lax.axis_index("c")
    s = lax.axis_index("s")
    tile = c * NS + s

    @pl.loop(0, CHUNK)
    def _(i):
        for k in range(D // 16):
            rows0[i, pl.ds(k * 16, 16)] = jnp.zeros((16,), jnp.float32)

    def unpack(j, slot):
        for k in range(CHUNK // 16):
            p = pidx_v[j, pl.ds(k * 16, 16)]
            ring[2 * slot, pl.ds(k * 16, 16)] = lax.shift_right_logical(p, 14)
            ring[2 * slot + 1, pl.ds(k * 16, 16)] = lax.bitwise_and(p, 16383)

    row0 = s * RPT
    for k in range(RPT // CHUNK):
        pltpu.sync_copy(rows0, acc.at[pl.ds(row0 + k * CHUNK, CHUNK)])

    pltpu.sync_copy(pidx_hbm.at[tile], pidx_v)

    plsc.subcore_barrier()

    unpack(0, 0)
    pltpu.async_copy(h_hbm.at[ring.at[0]], rows0, sem0)

    @pl.loop(0, NCH - 1, step=2)
    def _(j):
        unpack(j + 1, 1)
        cp1 = pltpu.async_copy(h_hbm.at[ring.at[2]], rows1, sem1)
        pltpu.make_async_copy(h_hbm.at[ring.at[0]], rows0, sem0).wait()
        pltpu.sync_copy(rows0, acc.at[ring.at[1]], add=True)

        unpack(j + 2, 0)
        pltpu.async_copy(h_hbm.at[ring.at[0]], rows0, sem0)

        cp1.wait()
        pltpu.sync_copy(rows1, acc.at[ring.at[3]], add=True)

    pltpu.make_async_copy(h_hbm.at[ring.at[0]], rows0, sem0).wait()
    pltpu.sync_copy(rows0, acc.at[ring.at[1]], add=True)

    plsc.subcore_barrier()

    pltpu.sync_copy(acc.at[pl.ds(row0, RPT)], out_hbm.at[c, pl.ds(row0, RPT)])


_sc_segsum = pl.kernel(
    _sc_segsum_body,
    out_type=jax.ShapeDtypeStruct((NC, NPAD, D), jnp.float32),
    mesh=plsc.VectorSubcoreMesh(core_axis_name="c", subcore_axis_name="s"),
    scratch_types=[
        pltpu.VMEM((NCH, CHUNK), jnp.int32),
        pltpu.VMEM((4, CHUNK), jnp.int32),
        pltpu.VMEM((CHUNK, D), jnp.float32),
        pltpu.VMEM((CHUNK, D), jnp.float32),
        pltpu.VMEM_SHARED((NPAD, D), jnp.float32),
        pltpu.SemaphoreType.DMA,
        pltpu.SemaphoreType.DMA,
    ],
)


BLK = 1000


def _tc_root_body(h_ref, wroot_t, b_ref, o_ref):
    o_ref[...] = jnp.dot(h_ref[...], wroot_t[...],
                         preferred_element_type=jnp.float32) + b_ref[...]


def _tc_root(h, wroot_t, b2d):
    return pl.pallas_call(
        _tc_root_body,
        grid=(N // BLK,),
        in_specs=[
            pl.BlockSpec((BLK, D), lambda i: (i, 0)),
            pl.BlockSpec((D, D), lambda i: (0, 0)),
            pl.BlockSpec((1, D), lambda i: (0, 0)),
        ],
        out_specs=pl.BlockSpec((BLK, D), lambda i: (i, 0)),
        out_shape=jax.ShapeDtypeStruct((N, D), jnp.float32),
    )(h, wroot_t, b2d)


def _tc_rel_body(p_ref, r_ref, wrel_t, flag_ref, o_ref):
    a = p_ref[0] + p_ref[1]
    acc = jnp.dot(a, wrel_t[...], preferred_element_type=jnp.float32)
    acc = acc + r_ref[...]
    acc = jnp.where(flag_ref[...] > 0.5, jnp.maximum(acc, 0.0), acc)
    o_ref[...] = acc


def _tc_rel(p, r, wrel_t, flag2d):
    return pl.pallas_call(
        _tc_rel_body,
        grid=(N // BLK,),
        in_specs=[
            pl.BlockSpec((NC, BLK, D), lambda i: (0, i, 0)),
            pl.BlockSpec((BLK, D), lambda i: (i, 0)),
            pl.BlockSpec((D, D), lambda i: (0, 0)),
            pl.BlockSpec((1, D), lambda i: (0, 0)),
        ],
        out_specs=pl.BlockSpec((BLK, D), lambda i: (i, 0)),
        out_shape=jax.ShapeDtypeStruct((N, D), jnp.float32),
    )(p, r, wrel_t, flag2d)


def kernel(x, edge_index, W1_rel, W1_root, b1, W2_rel, W2_root, b2,
           W3_rel, W3_root, b3):
    src = edge_index[0]
    dst = edge_index[1]
    packed = jnp.left_shift(src, 14) | dst
    pidx = packed.reshape(NTILES, NCH, CHUNK)

    wrel_t = jnp.stack([W1_rel.T, W2_rel.T, W3_rel.T])
    wroot_t = jnp.stack([W1_root.T, W2_root.T, W3_root.T])
    bias = jnp.stack([b1.reshape(1, D), b2.reshape(1, D), b3.reshape(1, D)])
    flags = jnp.stack([jnp.full((1, D), f, jnp.float32) for f in (1., 1., 0.)])

    def layer(h, xs):
        wr, wo, b2d_, fl = xs
        p = _sc_segsum(h, pidx)
        r = _tc_root(h, wo, b2d_)
        return _tc_rel(p, r, wr, fl), None

    h, _ = lax.scan(layer, x, (wrel_t, wroot_t, bias, flags))
    return h.reshape(-1)


# speedup vs baseline: 1.1185x; 1.1185x over previous
import functools

import jax
import jax.numpy as jnp
from jax import lax
from jax.experimental import pallas as pl
from jax.experimental.pallas import tpu as pltpu
from jax.experimental.pallas import tpu_sc as plsc

N = 10000
D = 128
E = 320000

NC = 2
NS = 16
NTILES = NC * NS
CHUNK = 128
NCH = 80
EPT = NCH * CHUNK
EPAD = NTILES * EPT
NPAD = 10240
RPT = NPAD // NS


def _sc_segsum_body(h_hbm, pidx_hbm, out_hbm,
                    pidx_v, ring, rows0, rows1, acc, sem0, sem1):
    c = lax.axis_index("c")
    s = lax.axis_index("s")
    tile = c * NS + s

    @pl.loop(0, CHUNK)
    def _(i):
        for k in range(D // 16):
            rows0[i, pl.ds(k * 16, 16)] = jnp.zeros((16,), jnp.float32)

    def unpack(j, slot):
        for k in range(CHUNK // 16):
            p = pidx_v[j, pl.ds(k * 16, 16)]
            ring[2 * slot, pl.ds(k * 16, 16)] = lax.shift_right_logical(p, 14)
            ring[2 * slot + 1, pl.ds(k * 16, 16)] = lax.bitwise_and(p, 16383)

    row0 = s * RPT
    for k in range(RPT // CHUNK):
        pltpu.sync_copy(rows0, acc.at[pl.ds(row0 + k * CHUNK, CHUNK)])

    pltpu.sync_copy(pidx_hbm.at[tile], pidx_v)

    plsc.subcore_barrier()

    unpack(0, 0)
    pltpu.async_copy(h_hbm.at[ring.at[0]], rows0, sem0)

    @pl.loop(0, NCH, step=2)
    def _(j):
        unpack(j + 1, 1)
        cp1 = pltpu.async_copy(h_hbm.at[ring.at[2]], rows1, sem1)
        pltpu.make_async_copy(h_hbm.at[ring.at[0]], rows0, sem0).wait()
        pltpu.sync_copy(rows0, acc.at[ring.at[1]], add=True)

        @pl.when(j + 2 < NCH)
        def _():
            unpack(j + 2, 0)
            pltpu.async_copy(h_hbm.at[ring.at[0]], rows0, sem0)

        cp1.wait()
        pltpu.sync_copy(rows1, acc.at[ring.at[3]], add=True)

    plsc.subcore_barrier()

    pltpu.sync_copy(acc.at[pl.ds(row0, RPT)], out_hbm.at[c, pl.ds(row0, RPT)])


_sc_segsum = pl.kernel(
    _sc_segsum_body,
    out_type=jax.ShapeDtypeStruct((NC, NPAD, D), jnp.float32),
    mesh=plsc.VectorSubcoreMesh(core_axis_name="c", subcore_axis_name="s"),
    scratch_types=[
        pltpu.VMEM((NCH, CHUNK), jnp.int32),
        pltpu.VMEM((4, CHUNK), jnp.int32),
        pltpu.VMEM((CHUNK, D), jnp.float32),
        pltpu.VMEM((CHUNK, D), jnp.float32),
        pltpu.VMEM_SHARED((NPAD, D), jnp.float32),
        pltpu.SemaphoreType.DMA,
        pltpu.SemaphoreType.DMA,
    ],
)


BLK = 1000


def _tc_root_body(h_ref, wroot_t, b_ref, o_ref):
    o_ref[...] = jnp.dot(h_ref[...], wroot_t[...],
                         preferred_element_type=jnp.float32) + b_ref[...]


def _tc_root(h, wroot_t, b2d):
    return pl.pallas_call(
        _tc_root_body,
        grid=(N // BLK,),
        in_specs=[
            pl.BlockSpec((BLK, D), lambda i: (i, 0)),
            pl.BlockSpec((D, D), lambda i: (0, 0)),
            pl.BlockSpec((1, D), lambda i: (0, 0)),
        ],
        out_specs=pl.BlockSpec((BLK, D), lambda i: (i, 0)),
        out_shape=jax.ShapeDtypeStruct((N, D), jnp.float32),
    )(h, wroot_t, b2d)


def _tc_rel_body(p_ref, r_ref, wrel_t, flag_ref, o_ref):
    a = p_ref[0] + p_ref[1]
    acc = jnp.dot(a, wrel_t[...], preferred_element_type=jnp.float32)
    acc = acc + r_ref[...]
    acc = jnp.where(flag_ref[...] > 0.5, jnp.maximum(acc, 0.0), acc)
    o_ref[...] = acc


def _tc_rel(p, r, wrel_t, flag2d):
    return pl.pallas_call(
        _tc_rel_body,
        grid=(N // BLK,),
        in_specs=[
            pl.BlockSpec((NC, BLK, D), lambda i: (0, i, 0)),
            pl.BlockSpec((BLK, D), lambda i: (i, 0)),
            pl.BlockSpec((D, D), lambda i: (0, 0)),
            pl.BlockSpec((1, D), lambda i: (0, 0)),
        ],
        out_specs=pl.BlockSpec((BLK, D), lambda i: (i, 0)),
        out_shape=jax.ShapeDtypeStruct((N, D), jnp.float32),
    )(p, r, wrel_t, flag2d)


def _tc_pack_body(ei_ref, o_ref):
    p = jnp.left_shift(ei_ref[0], 14) | ei_ref[1]
    ar = lax.iota(jnp.int32, EPAD - E) % (NPAD - N)
    junk = jnp.left_shift(ar, 14) | (N + ar)
    o_ref[...] = jnp.concatenate([p, junk]).reshape(NTILES, NCH, CHUNK)


def _tc_pack(edge_index):
    return pl.pallas_call(
        _tc_pack_body,
        out_shape=jax.ShapeDtypeStruct((NTILES, NCH, CHUNK), jnp.int32),
    )(edge_index)


def kernel(x, edge_index, W1_rel, W1_root, b1, W2_rel, W2_root, b2,
           W3_rel, W3_root, b3):
    pidx = _tc_pack(edge_index)

    wrel_t = jnp.stack([W1_rel.T, W2_rel.T, W3_rel.T])
    wroot_t = jnp.stack([W1_root.T, W2_root.T, W3_root.T])
    bias = jnp.stack([b1.reshape(1, D), b2.reshape(1, D), b3.reshape(1, D)])
    flags = jnp.stack([jnp.full((1, D), f, jnp.float32) for f in (1., 1., 0.)])

    def layer(h, xs):
        wr, wo, b2d_, fl = xs
        p = _sc_segsum(h, pidx)
        r = _tc_root(h, wo, b2d_)
        return _tc_rel(p, r, wr, fl), None

    h, _ = lax.scan(layer, x, (wrel_t, wroot_t, bias, flags))
    return h.reshape(-1)


# speedup vs baseline: 1.1441x; 1.0229x over previous
import functools

import jax
import jax.numpy as jnp
from jax import lax
from jax.experimental import pallas as pl
from jax.experimental.pallas import tpu as pltpu
from jax.experimental.pallas import tpu_sc as plsc

N = 10000
D = 128
E = 320000

NC = 2
NS = 16
NTILES = NC * NS
CHUNK = 128
NCH = 80
EPT = NCH * CHUNK
EPAD = NTILES * EPT
NPAD = 10240
RPT = NPAD // NS


def _sc_segsum_body(h_hbm, pidx_hbm, out_hbm,
                    pidx_v, ring, rows0, rows1, acc, sem0, sem1, semz):
    c = lax.axis_index("c")
    s = lax.axis_index("s")
    tile = c * NS + s

    cpi = pltpu.async_copy(pidx_hbm.at[tile], pidx_v, sem0)

    @pl.loop(0, CHUNK)
    def _(i):
        for k in range(D // 16):
            rows1[i, pl.ds(k * 16, 16)] = jnp.zeros((16,), jnp.float32)

    def unpack(j, slot):
        for k in range(CHUNK // 16):
            p = pidx_v[j, pl.ds(k * 16, 16)]
            ring[2 * slot, pl.ds(k * 16, 16)] = lax.shift_right_logical(p, 14)
            ring[2 * slot + 1, pl.ds(k * 16, 16)] = lax.bitwise_and(p, 16383)

    row0 = s * RPT
    zeros = [
        pltpu.async_copy(rows1, acc.at[pl.ds(row0 + k * CHUNK, CHUNK)], semz)
        for k in range(RPT // CHUNK)
    ]

    cpi.wait()
    unpack(0, 0)
    pltpu.async_copy(h_hbm.at[ring.at[0]], rows0, sem0)

    for cp in zeros:
        cp.wait()
    plsc.subcore_barrier()

    @pl.loop(0, NCH, step=2)
    def _(j):
        unpack(j + 1, 1)
        cp1 = pltpu.async_copy(h_hbm.at[ring.at[2]], rows1, sem1)
        pltpu.make_async_copy(h_hbm.at[ring.at[0]], rows0, sem0).wait()
        pltpu.sync_copy(rows0, acc.at[ring.at[1]], add=True)

        @pl.when(j + 2 < NCH)
        def _():
            unpack(j + 2, 0)
            pltpu.async_copy(h_hbm.at[ring.at[0]], rows0, sem0)

        cp1.wait()
        pltpu.sync_copy(rows1, acc.at[ring.at[3]], add=True)

    plsc.subcore_barrier()

    pltpu.sync_copy(acc.at[pl.ds(row0, RPT)], out_hbm.at[c, pl.ds(row0, RPT)])


_sc_segsum = pl.kernel(
    _sc_segsum_body,
    out_type=jax.ShapeDtypeStruct((NC, NPAD, D), jnp.float32),
    mesh=plsc.VectorSubcoreMesh(core_axis_name="c", subcore_axis_name="s"),
    scratch_types=[
        pltpu.VMEM((NCH, CHUNK), jnp.int32),
        pltpu.VMEM((4, CHUNK), jnp.int32),
        pltpu.VMEM((CHUNK, D), jnp.float32),
        pltpu.VMEM((CHUNK, D), jnp.float32),
        pltpu.VMEM_SHARED((NPAD, D), jnp.float32),
        pltpu.SemaphoreType.DMA,
        pltpu.SemaphoreType.DMA,
        pltpu.SemaphoreType.DMA,
    ],
)


BLK = 1000


def _tc_root_body(h_ref, wroot_t, b_ref, o_ref):
    o_ref[...] = jnp.dot(h_ref[...], wroot_t[...],
                         preferred_element_type=jnp.float32) + b_ref[...]


def _tc_root(h, wroot_t, b2d):
    return pl.pallas_call(
        _tc_root_body,
        grid=(N // BLK,),
        in_specs=[
            pl.BlockSpec((BLK, D), lambda i: (i, 0)),
            pl.BlockSpec((D, D), lambda i: (0, 0)),
            pl.BlockSpec((1, D), lambda i: (0, 0)),
        ],
        out_specs=pl.BlockSpec((BLK, D), lambda i: (i, 0)),
        out_shape=jax.ShapeDtypeStruct((N, D), jnp.float32),
    )(h, wroot_t, b2d)


def _tc_rel_body(p_ref, r_ref, wrel_t, flag_ref, o_ref):
    a = p_ref[0] + p_ref[1]
    acc = jnp.dot(a, wrel_t[...], preferred_element_type=jnp.float32)
    acc = acc + r_ref[...]
    acc = jnp.where(flag_ref[...] > 0.5, jnp.maximum(acc, 0.0), acc)
    o_ref[...] = acc


def _tc_rel(p, r, wrel_t, flag2d):
    return pl.pallas_call(
        _tc_rel_body,
        grid=(N // BLK,),
        in_specs=[
            pl.BlockSpec((NC, BLK, D), lambda i: (0, i, 0)),
            pl.BlockSpec((BLK, D), lambda i: (i, 0)),
            pl.BlockSpec((D, D), lambda i: (0, 0)),
            pl.BlockSpec((1, D), lambda i: (0, 0)),
        ],
        out_specs=pl.BlockSpec((BLK, D), lambda i: (i, 0)),
        out_shape=jax.ShapeDtypeStruct((N, D), jnp.float32),
    )(p, r, wrel_t, flag2d)


def _tc_pack_body(ei_ref, o_ref):
    p = jnp.left_shift(ei_ref[0], 14) | ei_ref[1]
    ar = lax.iota(jnp.int32, EPAD - E) % (NPAD - N)
    junk = jnp.left_shift(ar, 14) | (N + ar)
    o_ref[...] = jnp.concatenate([p, junk]).reshape(NTILES, NCH, CHUNK)


def _tc_pack(edge_index):
    return pl.pallas_call(
        _tc_pack_body,
        out_shape=jax.ShapeDtypeStruct((NTILES, NCH, CHUNK), jnp.int32),
    )(edge_index)


def kernel(x, edge_index, W1_rel, W1_root, b1, W2_rel, W2_root, b2,
           W3_rel, W3_root, b3):
    pidx = _tc_pack(edge_index)

    wrel_t = jnp.stack([W1_rel.T, W2_rel.T, W3_rel.T])
    wroot_t = jnp.stack([W1_root.T, W2_root.T, W3_root.T])
    bias = jnp.stack([b1.reshape(1, D), b2.reshape(1, D), b3.reshape(1, D)])
    flags = jnp.stack([jnp.full((1, D), f, jnp.float32) for f in (1., 1., 0.)])

    def layer(h, xs):
        wr, wo, b2d_, fl = xs
        p = _sc_segsum(h, pidx)
        r = _tc_root(h, wo, b2d_)
        return _tc_rel(p, r, wr, fl), None

    h, _ = lax.scan(layer, x, (wrel_t, wroot_t, bias, flags))
    return h.reshape(-1)


# speedup vs baseline: 1.1516x; 1.0066x over previous
import functools

import jax
import jax.numpy as jnp
from jax import lax
from jax.experimental import pallas as pl
from jax.experimental.pallas import tpu as pltpu
from jax.experimental.pallas import tpu_sc as plsc

N = 10000
D = 128
E = 320000

NC = 2
NS = 16
NTILES = NC * NS
CHUNK = 128
NCH = 80
EPT = NCH * CHUNK
EPAD = NTILES * EPT
NPAD = 10240
RPT = NPAD // NS


def _sc_segsum_body(h_hbm, pidx_hbm, out_hbm,
                    pidx_v, ring, rows0, rows1, acc, sem0, sem1, semz):
    c = lax.axis_index("c")
    s = lax.axis_index("s")
    tile = c * NS + s

    cpi = pltpu.async_copy(pidx_hbm.at[tile], pidx_v, sem0)

    @pl.loop(0, CHUNK)
    def _(i):
        for k in range(D // 16):
            rows1[i, pl.ds(k * 16, 16)] = jnp.zeros((16,), jnp.float32)

    def unpack(j, slot):
        for k in range(CHUNK // 16):
            p = pidx_v[j, pl.ds(k * 16, 16)]
            ring[2 * slot, pl.ds(k * 16, 16)] = lax.shift_right_logical(p, 14)
            ring[2 * slot + 1, pl.ds(k * 16, 16)] = lax.bitwise_and(p, 16383)

    row0 = s * RPT
    zeros = [
        pltpu.async_copy(rows1, acc.at[pl.ds(row0 + k * CHUNK, CHUNK)], semz)
        for k in range(RPT // CHUNK)
    ]

    cpi.wait()
    unpack(0, 0)
    pltpu.async_copy(h_hbm.at[ring.at[0]], rows0, sem0)

    for cp in zeros:
        cp.wait()
    plsc.subcore_barrier()

    @pl.loop(0, NCH, step=2)
    def _(j):
        unpack(j + 1, 1)
        cp1 = pltpu.async_copy(h_hbm.at[ring.at[2]], rows1, sem1)
        pltpu.make_async_copy(h_hbm.at[ring.at[0]], rows0, sem0).wait()
        pltpu.sync_copy(rows0, acc.at[ring.at[1]], add=True)

        @pl.when(j + 2 < NCH)
        def _():
            unpack(j + 2, 0)
            pltpu.async_copy(h_hbm.at[ring.at[0]], rows0, sem0)

        cp1.wait()
        pltpu.sync_copy(rows1, acc.at[ring.at[3]], add=True)

    plsc.subcore_barrier()

    pltpu.sync_copy(acc.at[pl.ds(row0, RPT)], out_hbm.at[c, pl.ds(row0, RPT)])


_sc_segsum = pl.kernel(
    _sc_segsum_body,
    out_type=jax.ShapeDtypeStruct((NC, NPAD, D), jnp.float32),
    mesh=plsc.VectorSubcoreMesh(core_axis_name="c", subcore_axis_name="s"),
    scratch_types=[
        pltpu.VMEM((NCH, CHUNK), jnp.int32),
        pltpu.VMEM((4, CHUNK), jnp.int32),
        pltpu.VMEM((CHUNK, D), jnp.float32),
        pltpu.VMEM((CHUNK, D), jnp.float32),
        pltpu.VMEM_SHARED((NPAD, D), jnp.float32),
        pltpu.SemaphoreType.DMA,
        pltpu.SemaphoreType.DMA,
        pltpu.SemaphoreType.DMA,
    ],
)


BLK = 1000


def _tc_root_body(h_ref, wroot, b_ref, o_ref):
    o_ref[...] = lax.dot_general(
        h_ref[...], wroot[...], (((1,), (1,)), ((), ())),
        preferred_element_type=jnp.float32) + b_ref[...]


def _tc_root(h, wroot_t, b2d):
    return pl.pallas_call(
        _tc_root_body,
        grid=(N // BLK,),
        in_specs=[
            pl.BlockSpec((BLK, D), lambda i: (i, 0)),
            pl.BlockSpec((D, D), lambda i: (0, 0)),
            pl.BlockSpec((1, D), lambda i: (0, 0)),
        ],
        out_specs=pl.BlockSpec((BLK, D), lambda i: (i, 0)),
        out_shape=jax.ShapeDtypeStruct((N, D), jnp.float32),
    )(h, wroot_t, b2d)


def _tc_rel_body(p_ref, r_ref, wrel, flag_ref, o_ref):
    a = p_ref[0] + p_ref[1]
    acc = lax.dot_general(a, wrel[...], (((1,), (1,)), ((), ())),
                          preferred_element_type=jnp.float32)
    acc = acc + r_ref[...]
    acc = jnp.where(flag_ref[...] > 0.5, jnp.maximum(acc, 0.0), acc)
    o_ref[...] = acc


def _tc_rel(p, r, wrel_t, flag2d):
    return pl.pallas_call(
        _tc_rel_body,
        grid=(N // BLK,),
        in_specs=[
            pl.BlockSpec((NC, BLK, D), lambda i: (0, i, 0)),
            pl.BlockSpec((BLK, D), lambda i: (i, 0)),
            pl.BlockSpec((D, D), lambda i: (0, 0)),
            pl.BlockSpec((1, D), lambda i: (0, 0)),
        ],
        out_specs=pl.BlockSpec((BLK, D), lambda i: (i, 0)),
        out_shape=jax.ShapeDtypeStruct((N, D), jnp.float32),
    )(p, r, wrel_t, flag2d)


def _tc_pack_body(ei_ref, o_ref):
    p = jnp.left_shift(ei_ref[0], 14) | ei_ref[1]
    ar = lax.iota(jnp.int32, EPAD - E) % (NPAD - N)
    junk = jnp.left_shift(ar, 14) | (N + ar)
    o_ref[...] = jnp.concatenate([p, junk]).reshape(NTILES, NCH, CHUNK)


def _tc_pack(edge_index):
    return pl.pallas_call(
        _tc_pack_body,
        out_shape=jax.ShapeDtypeStruct((NTILES, NCH, CHUNK), jnp.int32),
    )(edge_index)


def kernel(x, edge_index, W1_rel, W1_root, b1, W2_rel, W2_root, b2,
           W3_rel, W3_root, b3):
    pidx = _tc_pack(edge_index)

    wrel_s = jnp.stack([W1_rel, W2_rel, W3_rel])
    wroot_s = jnp.stack([W1_root, W2_root, W3_root])
    bias = jnp.stack([b1.reshape(1, D), b2.reshape(1, D), b3.reshape(1, D)])
    flags = jnp.stack([jnp.full((1, D), f, jnp.float32) for f in (1., 1., 0.)])

    def layer(h, xs):
        wr, wo, b2d_, fl = xs
        p = _sc_segsum(h, pidx)
        r = _tc_root(h, wo, b2d_)
        return _tc_rel(p, r, wr, fl), None

    h, _ = lax.scan(layer, x, (wrel_s, wroot_s, bias, flags))
    return h.reshape(-1)
